# baseline (speedup 1.0000x reference)
import functools

import jax
import jax.numpy as jnp
from jax import lax
from jax.experimental import pallas as pl
from jax.experimental.pallas import tpu as pltpu
from jax.experimental.pallas import tpu_sc as plsc

T = 2048
D = 1024
H = 2048
O = 1024
E = 8
K = 2

BT = 512
NE = T * K
NB = (NE + E * (BT - 1) + BT - 1) // BT
P = NB * BT

NWORKERS = 32


def _gating_body(x_ref, Wg_ref, bg_ref, ti_ref, tw_ref):
    logits = jnp.dot(x_ref[...], Wg_ref[...],
                     preferred_element_type=jnp.float32) + bg_ref[...]
    idx = lax.broadcasted_iota(jnp.int32, (T, E), 1)
    m1 = jnp.max(logits, axis=1, keepdims=True)
    i1 = jnp.min(jnp.where(logits >= m1, idx, E), axis=1, keepdims=True)
    rest = jnp.where(idx == i1, -jnp.inf, logits)
    m2 = jnp.max(rest, axis=1, keepdims=True)
    i2 = jnp.min(jnp.where(rest >= m2, idx, E), axis=1, keepdims=True)
    r = jnp.exp(m2 - m1)
    w1 = 1.0 / (1.0 + r)
    ti_ref[...] = jnp.concatenate([i1, i2], axis=1)
    tw_ref[...] = jnp.concatenate([w1, 1.0 - w1], axis=1)


def _gating(x, Wg, bg):
    return pl.pallas_call(
        _gating_body,
        out_shape=(jax.ShapeDtypeStruct((T, K), jnp.int32),
                   jax.ShapeDtypeStruct((T, K), jnp.float32)),
    )(x, Wg, bg.reshape(1, E))


def _route_meta(top_i):
    f = top_i.T.reshape(-1)
    oh = (f[:, None] == jnp.arange(E, dtype=jnp.int32)[None, :])
    ohi = oh.astype(jnp.int32)
    ranks = jnp.cumsum(ohi, axis=0) - ohi
    rank = jnp.sum(ranks * ohi, axis=1)
    counts = jnp.sum(ohi, axis=0)
    nblk = (counts + BT - 1) // BT
    blk_end = jnp.cumsum(nblk)
    blk_start = jnp.concatenate([jnp.zeros((1,), jnp.int32), blk_end[:-1]])
    dest = (blk_start[f] * BT + rank).astype(jnp.int32)

    nb_used = blk_end[E - 1]
    bidx = jnp.arange(NB, dtype=jnp.int32)
    be = jnp.searchsorted(blk_end, bidx, side="right").astype(jnp.int32)
    be_last = be[nb_used - 1]
    block_expert = jnp.where(bidx < nb_used, be, be_last)
    block_valid = (bidx < nb_used).astype(jnp.int32)
    block_x = jnp.where(bidx < nb_used, bidx, nb_used - 1)
    return dest, block_expert, block_valid, block_x


def _make_sc_gather(n_rows, d, chunk):
    per_w = n_rows // NWORKERS
    n_chunks = per_w // chunk
    mesh = plsc.VectorSubcoreMesh(core_axis_name="c", subcore_axis_name="s")

    def body(src_hbm, idx_hbm, out_hbm, idx_v, rows_v, sem):
        wid = lax.axis_index("s") * 2 + lax.axis_index("c")
        base = wid * per_w

        def step(i, carry):
            off = base + i * chunk
            pltpu.sync_copy(idx_hbm.at[pl.ds(off, chunk)], idx_v)
            pltpu.async_copy(src_hbm.at[idx_v], rows_v, sem).wait()
            pltpu.sync_copy(rows_v, out_hbm.at[pl.ds(off, chunk)])
            return carry

        lax.fori_loop(0, n_chunks, step, 0)

    return pl.kernel(
        body,
        out_type=jax.ShapeDtypeStruct((n_rows, d), jnp.float32),
        mesh=mesh,
        scratch_types=[
            pltpu.VMEM((chunk,), jnp.int32),
            pltpu.VMEM((chunk, d), jnp.float32),
            pltpu.SemaphoreType.DMA,
        ],
    )


@functools.cache
def _sc_gather(n_rows, d, chunk):
    return _make_sc_gather(n_rows, d, chunk)


def _sc_gather_y(src, idx):
    return _sc_gather(NE, O, 64)(src, idx)


def _make_sc_dispatch(chunk):
    per_w = NE // NWORKERS
    n_chunks = per_w // chunk
    mesh = plsc.VectorSubcoreMesh(core_axis_name="c", subcore_axis_name="s")

    def body(x_hbm, dest_hbm, tok_hbm, xs_hbm, tok_v, dest_v, rows_v, sem):
        wid = lax.axis_index("s") * 2 + lax.axis_index("c")
        base = wid * per_w

        def step(i, carry):
            off = base + i * chunk
            pltpu.sync_copy(tok_hbm.at[pl.ds(off, chunk)], tok_v)
            pltpu.sync_copy(dest_hbm.at[pl.ds(off, chunk)], dest_v)
            pltpu.async_copy(x_hbm.at[tok_v], rows_v, sem).wait()
            pltpu.async_copy(rows_v, xs_hbm.at[dest_v], sem).wait()
            return carry

        lax.fori_loop(0, n_chunks, step, 0)

    return pl.kernel(
        body,
        out_type=jax.ShapeDtypeStruct((P, D), jnp.float32),
        mesh=mesh,
        scratch_types=[
            pltpu.VMEM((chunk,), jnp.int32),
            pltpu.VMEM((chunk,), jnp.int32),
            pltpu.VMEM((chunk, D), jnp.float32),
            pltpu.SemaphoreType.DMA,
        ],
    )


@functools.cache
def _sc_dispatch(chunk):
    return _make_sc_dispatch(chunk)


def _sc_dispatch_x(x, dest):
    tok = jnp.tile(jnp.arange(T, dtype=jnp.int32), K)
    return _sc_dispatch(64)(x, dest, tok)


NJ = 2
H2 = H // NJ


def _grouped_body(be_ref, valid_ref, bx_ref, xs_ref, W1_ref, b1_ref, W2_ref,
                  b2_ref, out_ref):
    b = pl.program_id(0)
    j = pl.program_id(1)

    @pl.when(valid_ref[b] == 1)
    def _():
        h = jnp.maximum(
            jnp.dot(xs_ref[...], W1_ref[0],
                    preferred_element_type=jnp.float32) + b1_ref[0], 0.0)
        y = jnp.dot(h, W2_ref[0], preferred_element_type=jnp.float32)

        @pl.when(j == 0)
        def _():
            out_ref[...] = y + b2_ref[0]

        @pl.when(j != 0)
        def _():
            out_ref[...] += y


def _grouped_mlp(xs, W1, b1, W2, b2, block_expert, block_valid, block_x):
    grid_spec = pltpu.PrefetchScalarGridSpec(
        num_scalar_prefetch=3,
        grid=(NB, NJ),
        in_specs=[
            pl.BlockSpec((BT, D), lambda b, j, be, v, bx: (bx[b], 0)),
            pl.BlockSpec((1, D, H2), lambda b, j, be, v, bx: (be[b], 0, j)),
            pl.BlockSpec((1, 1, H2), lambda b, j, be, v, bx: (be[b], 0, j)),
            pl.BlockSpec((1, H2, O), lambda b, j, be, v, bx: (be[b], j, 0)),
            pl.BlockSpec((1, 1, O), lambda b, j, be, v, bx: (be[b], 0, 0)),
        ],
        out_specs=pl.BlockSpec((BT, O), lambda b, j, be, v, bx: (b, 0)),
    )
    return pl.pallas_call(
        _grouped_body,
        grid_spec=grid_spec,
        out_shape=jax.ShapeDtypeStruct((P, O), jnp.float32),
    )(block_expert, block_valid, block_x, xs, W1, b1.reshape(E, 1, H), W2,
      b2.reshape(E, 1, O))


def _combine_body(g0_ref, g1_ref, tw_ref, out_ref):
    w0 = tw_ref[:, 0:1]
    w1 = tw_ref[:, 1:2]
    out_ref[...] = w0 * g0_ref[...] + w1 * g1_ref[...]


def _combine(g, top_w):
    btc = 512
    nblk = T // btc
    return pl.pallas_call(
        _combine_body,
        grid=(nblk,),
        in_specs=[pl.BlockSpec((btc, O), lambda t: (t, 0)),
                  pl.BlockSpec((btc, O), lambda t, n=nblk: (t + n, 0)),
                  pl.BlockSpec((btc, K), lambda t: (t, 0))],
        out_specs=pl.BlockSpec((btc, O), lambda t: (t, 0)),
        out_shape=jax.ShapeDtypeStruct((T, O), jnp.float32),
    )(g, g, top_w)


@jax.jit
def kernel(x, Wg, bg, W1, b1, W2, b2):
    top_i, top_w = _gating(x, Wg, bg)
    dest, block_expert, block_valid, block_x = _route_meta(top_i)
    xs = _sc_dispatch_x(x, dest)
    ys = _grouped_mlp(xs, W1, b1, W2, b2, block_expert, block_valid, block_x)
    g = _sc_gather_y(ys, dest)
    return _combine(g, top_w)


# speedup vs baseline: 1.1946x; 1.1946x over previous
import functools

import jax
import jax.numpy as jnp
from jax import lax
from jax.experimental import pallas as pl
from jax.experimental.pallas import tpu as pltpu
from jax.experimental.pallas import tpu_sc as plsc

T = 2048
D = 1024
H = 2048
O = 1024
E = 8
K = 2

BT = 512
NE = T * K
NB = (NE + E * (BT - 1) + BT - 1) // BT
P = NB * BT

NWORKERS = 32


def _gating_body(x_ref, Wg_ref, bg_ref, ti_ref, tw_ref):
    logits = jnp.dot(x_ref[...], Wg_ref[...],
                     preferred_element_type=jnp.float32) + bg_ref[...]
    idx = lax.broadcasted_iota(jnp.int32, (T, E), 1)
    m1 = jnp.max(logits, axis=1, keepdims=True)
    i1 = jnp.min(jnp.where(logits >= m1, idx, E), axis=1, keepdims=True)
    rest = jnp.where(idx == i1, -jnp.inf, logits)
    m2 = jnp.max(rest, axis=1, keepdims=True)
    i2 = jnp.min(jnp.where(rest >= m2, idx, E), axis=1, keepdims=True)
    r = jnp.exp(m2 - m1)
    w1 = 1.0 / (1.0 + r)
    ti_ref[...] = jnp.concatenate([i1, i2], axis=1)
    tw_ref[...] = jnp.concatenate([w1, 1.0 - w1], axis=1)


def _gating(x, Wg, bg):
    return pl.pallas_call(
        _gating_body,
        out_shape=(jax.ShapeDtypeStruct((T, K), jnp.int32),
                   jax.ShapeDtypeStruct((T, K), jnp.float32)),
    )(x, Wg, bg.reshape(1, E))


def _route_meta(top_i):
    f = top_i.T.reshape(-1)
    oh = (f[:, None] == jnp.arange(E, dtype=jnp.int32)[None, :])
    ohi = oh.astype(jnp.int32)
    ranks = jnp.cumsum(ohi, axis=0) - ohi
    rank = jnp.sum(ranks * ohi, axis=1)
    counts = jnp.sum(ohi, axis=0)
    nblk = (counts + BT - 1) // BT
    blk_end = jnp.cumsum(nblk)
    blk_start = jnp.concatenate([jnp.zeros((1,), jnp.int32), blk_end[:-1]])
    dest = (blk_start[f] * BT + rank).astype(jnp.int32)

    nb_used = blk_end[E - 1]
    bidx = jnp.arange(NB, dtype=jnp.int32)
    be = jnp.searchsorted(blk_end, bidx, side="right").astype(jnp.int32)
    be_last = be[nb_used - 1]
    block_expert = jnp.where(bidx < nb_used, be, be_last)
    block_valid = (bidx < nb_used).astype(jnp.int32)
    block_x = jnp.where(bidx < nb_used, bidx, nb_used - 1)
    return dest, block_expert, block_valid, block_x


def _make_sc_gather(n_rows, d, chunk):
    per_w = n_rows // NWORKERS
    n_chunks = per_w // chunk
    mesh = plsc.VectorSubcoreMesh(core_axis_name="c", subcore_axis_name="s")

    def body(src_hbm, idx_hbm, out_hbm, idx_v, rows_v, sem):
        wid = lax.axis_index("s") * 2 + lax.axis_index("c")
        base = wid * per_w

        def step(i, carry):
            off = base + i * chunk
            pltpu.sync_copy(idx_hbm.at[pl.ds(off, chunk)], idx_v)
            pltpu.async_copy(src_hbm.at[idx_v], rows_v, sem).wait()
            pltpu.sync_copy(rows_v, out_hbm.at[pl.ds(off, chunk)])
            return carry

        lax.fori_loop(0, n_chunks, step, 0)

    return pl.kernel(
        body,
        out_type=jax.ShapeDtypeStruct((n_rows, d), jnp.float32),
        mesh=mesh,
        scratch_types=[
            pltpu.VMEM((chunk,), jnp.int32),
            pltpu.VMEM((chunk, d), jnp.float32),
            pltpu.SemaphoreType.DMA,
        ],
    )


@functools.cache
def _sc_gather(n_rows, d, chunk):
    return _make_sc_gather(n_rows, d, chunk)


def _sc_gather_y(src, idx):
    return _sc_gather(NE, O, 64)(src, idx)


def _make_sc_dispatch(chunk):
    per_w = NE // NWORKERS
    n_chunks = per_w // chunk
    mesh = plsc.VectorSubcoreMesh(core_axis_name="c", subcore_axis_name="s")

    def body(x_hbm, dest_hbm, tok_hbm, xs_hbm, tok_v, dest_v, rows_v, sem):
        wid = lax.axis_index("s") * 2 + lax.axis_index("c")
        base = wid * per_w

        def step(i, carry):
            off = base + i * chunk
            pltpu.sync_copy(tok_hbm.at[pl.ds(off, chunk)], tok_v)
            pltpu.sync_copy(dest_hbm.at[pl.ds(off, chunk)], dest_v)
            pltpu.async_copy(x_hbm.at[tok_v], rows_v, sem).wait()
            pltpu.async_copy(rows_v, xs_hbm.at[dest_v], sem).wait()
            return carry

        lax.fori_loop(0, n_chunks, step, 0)

    return pl.kernel(
        body,
        out_type=jax.ShapeDtypeStruct((P, D), jnp.float32),
        mesh=mesh,
        scratch_types=[
            pltpu.VMEM((chunk,), jnp.int32),
            pltpu.VMEM((chunk,), jnp.int32),
            pltpu.VMEM((chunk, D), jnp.float32),
            pltpu.SemaphoreType.DMA,
        ],
    )


@functools.cache
def _sc_dispatch(chunk):
    return _make_sc_dispatch(chunk)


def _sc_dispatch_x(x, dest):
    tok = jnp.tile(jnp.arange(T, dtype=jnp.int32), K)
    return _sc_dispatch(64)(x, dest, tok)


NJ = 1
H2 = H // NJ


def _grouped_body(be_ref, valid_ref, bx_ref, xs_ref, W1_ref, b1_ref, W2_ref,
                  b2_ref, out_ref):
    b = pl.program_id(0)
    j = pl.program_id(1)

    @pl.when(valid_ref[b] == 1)
    def _():
        h = jnp.maximum(
            jnp.dot(xs_ref[...], W1_ref[0],
                    preferred_element_type=jnp.float32) + b1_ref[0], 0.0)
        y = jnp.dot(h, W2_ref[0], preferred_element_type=jnp.float32)

        @pl.when(j == 0)
        def _():
            out_ref[...] = y + b2_ref[0]

        @pl.when(j != 0)
        def _():
            out_ref[...] += y


def _grouped_mlp(xs, W1, b1, W2, b2, block_expert, block_valid, block_x):
    grid_spec = pltpu.PrefetchScalarGridSpec(
        num_scalar_prefetch=3,
        grid=(NB, NJ),
        in_specs=[
            pl.BlockSpec((BT, D), lambda b, j, be, v, bx: (bx[b], 0)),
            pl.BlockSpec((1, D, H2), lambda b, j, be, v, bx: (be[b], 0, j)),
            pl.BlockSpec((1, 1, H2), lambda b, j, be, v, bx: (be[b], 0, j)),
            pl.BlockSpec((1, H2, O), lambda b, j, be, v, bx: (be[b], j, 0)),
            pl.BlockSpec((1, 1, O), lambda b, j, be, v, bx: (be[b], 0, 0)),
        ],
        out_specs=pl.BlockSpec((BT, O), lambda b, j, be, v, bx: (b, 0)),
    )
    return pl.pallas_call(
        _grouped_body,
        grid_spec=grid_spec,
        out_shape=jax.ShapeDtypeStruct((P, O), jnp.float32),
    )(block_expert, block_valid, block_x, xs, W1, b1.reshape(E, 1, H), W2,
      b2.reshape(E, 1, O))


def _combine_body(g0_ref, g1_ref, tw_ref, out_ref):
    w0 = tw_ref[:, 0:1]
    w1 = tw_ref[:, 1:2]
    out_ref[...] = w0 * g0_ref[...] + w1 * g1_ref[...]


def _combine(g, top_w):
    btc = 512
    nblk = T // btc
    return pl.pallas_call(
        _combine_body,
        grid=(nblk,),
        in_specs=[pl.BlockSpec((btc, O), lambda t: (t, 0)),
                  pl.BlockSpec((btc, O), lambda t, n=nblk: (t + n, 0)),
                  pl.BlockSpec((btc, K), lambda t: (t, 0))],
        out_specs=pl.BlockSpec((btc, O), lambda t: (t, 0)),
        out_shape=jax.ShapeDtypeStruct((T, O), jnp.float32),
    )(g, g, top_w)


@jax.jit
def kernel(x, Wg, bg, W1, b1, W2, b2):
    top_i, top_w = _gating(x, Wg, bg)
    dest, block_expert, block_valid, block_x = _route_meta(top_i)
    xs = _sc_dispatch_x(x, dest)
    ys = _grouped_mlp(xs, W1, b1, W2, b2, block_expert, block_valid, block_x)
    g = _sc_gather_y(ys, dest)
    return _combine(g, top_w)


# speedup vs baseline: 1.1948x; 1.0001x over previous
import functools

import jax
import jax.numpy as jnp
from jax import lax
from jax.experimental import pallas as pl
from jax.experimental.pallas import tpu as pltpu
from jax.experimental.pallas import tpu_sc as plsc

T = 2048
D = 1024
H = 2048
O = 1024
E = 8
K = 2

BT = 512
NE = T * K
NB = (NE + E * (BT - 1) + BT - 1) // BT
P = NB * BT

NWORKERS = 32


def _gating_body(x_ref, Wg_ref, bg_ref, ti_ref, tw_ref):
    logits = jnp.dot(x_ref[...], Wg_ref[...],
                     preferred_element_type=jnp.float32) + bg_ref[...]
    idx = lax.broadcasted_iota(jnp.int32, (T, E), 1)
    m1 = jnp.max(logits, axis=1, keepdims=True)
    i1 = jnp.min(jnp.where(logits >= m1, idx, E), axis=1, keepdims=True)
    rest = jnp.where(idx == i1, -jnp.inf, logits)
    m2 = jnp.max(rest, axis=1, keepdims=True)
    i2 = jnp.min(jnp.where(rest >= m2, idx, E), axis=1, keepdims=True)
    r = jnp.exp(m2 - m1)
    w1 = 1.0 / (1.0 + r)
    ti_ref[...] = jnp.concatenate([i1, i2], axis=1)
    tw_ref[...] = jnp.concatenate([w1, 1.0 - w1], axis=1)


def _gating(x, Wg, bg):
    return pl.pallas_call(
        _gating_body,
        out_shape=(jax.ShapeDtypeStruct((T, K), jnp.int32),
                   jax.ShapeDtypeStruct((T, K), jnp.float32)),
    )(x, Wg, bg.reshape(1, E))


def _route_meta(top_i):
    f = top_i.T.reshape(-1)
    oh = (f[:, None] == jnp.arange(E, dtype=jnp.int32)[None, :])
    ohi = oh.astype(jnp.int32)
    ranks = jnp.cumsum(ohi, axis=0) - ohi
    rank = jnp.sum(ranks * ohi, axis=1)
    counts = jnp.sum(ohi, axis=0)
    nblk = (counts + BT - 1) // BT
    blk_end = jnp.cumsum(nblk)
    blk_start = jnp.concatenate([jnp.zeros((1,), jnp.int32), blk_end[:-1]])
    dest = (blk_start[f] * BT + rank).astype(jnp.int32)

    nb_used = blk_end[E - 1]
    bidx = jnp.arange(NB, dtype=jnp.int32)
    be = jnp.searchsorted(blk_end, bidx, side="right").astype(jnp.int32)
    be_last = be[nb_used - 1]
    block_expert = jnp.where(bidx < nb_used, be, be_last)
    block_valid = (bidx < nb_used).astype(jnp.int32)
    block_x = jnp.where(bidx < nb_used, bidx, nb_used - 1)
    return dest, block_expert, block_valid, block_x


def _make_sc_gather(n_rows, d, chunk):
    per_w = n_rows // NWORKERS
    n_chunks = per_w // chunk
    mesh = plsc.VectorSubcoreMesh(core_axis_name="c", subcore_axis_name="s")

    def body(src_hbm, idx_hbm, out_hbm, idx_v, rows_v, sem):
        wid = lax.axis_index("s") * 2 + lax.axis_index("c")
        base = wid * per_w

        def step(i, carry):
            off = base + i * chunk
            pltpu.sync_copy(idx_hbm.at[pl.ds(off, chunk)], idx_v)
            pltpu.async_copy(src_hbm.at[idx_v], rows_v, sem).wait()
            pltpu.sync_copy(rows_v, out_hbm.at[pl.ds(off, chunk)])
            return carry

        lax.fori_loop(0, n_chunks, step, 0)

    return pl.kernel(
        body,
        out_type=jax.ShapeDtypeStruct((n_rows, d), jnp.float32),
        mesh=mesh,
        scratch_types=[
            pltpu.VMEM((chunk,), jnp.int32),
            pltpu.VMEM((chunk, d), jnp.float32),
            pltpu.SemaphoreType.DMA,
        ],
    )


@functools.cache
def _sc_gather(n_rows, d, chunk):
    return _make_sc_gather(n_rows, d, chunk)


def _sc_gather_y(src, idx):
    return _sc_gather(NE, O, 64)(src, idx)


def _make_sc_dispatch(chunk):
    per_w = NE // NWORKERS
    n_chunks = per_w // chunk
    mesh = plsc.VectorSubcoreMesh(core_axis_name="c", subcore_axis_name="s")

    def body(x_hbm, dest_hbm, tok_hbm, xs_hbm, tok_v, dest_v, rows_v, sem):
        wid = lax.axis_index("s") * 2 + lax.axis_index("c")
        base = wid * per_w

        def step(i, carry):
            off = base + i * chunk
            pltpu.sync_copy(tok_hbm.at[pl.ds(off, chunk)], tok_v)
            pltpu.sync_copy(dest_hbm.at[pl.ds(off, chunk)], dest_v)
            pltpu.async_copy(x_hbm.at[tok_v], rows_v, sem).wait()
            pltpu.async_copy(rows_v, xs_hbm.at[dest_v], sem).wait()
            return carry

        lax.fori_loop(0, n_chunks, step, 0)

    return pl.kernel(
        body,
        out_type=jax.ShapeDtypeStruct((P, D), jnp.float32),
        mesh=mesh,
        scratch_types=[
            pltpu.VMEM((chunk,), jnp.int32),
            pltpu.VMEM((chunk,), jnp.int32),
            pltpu.VMEM((chunk, D), jnp.float32),
            pltpu.SemaphoreType.DMA,
        ],
    )


@functools.cache
def _sc_dispatch(chunk):
    return _make_sc_dispatch(chunk)


def _sc_dispatch_x(x, dest):
    tok = jnp.tile(jnp.arange(T, dtype=jnp.int32), K)
    return _sc_dispatch(64)(x, dest, tok)


NJ = 1
H2 = H // NJ


def _grouped_body(be_ref, valid_ref, bx_ref, xs_ref, W1_ref, b1_ref, W2_ref,
                  b2_ref, out_ref):
    b = pl.program_id(0)
    j = pl.program_id(1)

    @pl.when(valid_ref[b] == 1)
    def _():
        xb = xs_ref[...].astype(jnp.bfloat16)
        h = jnp.maximum(
            jnp.dot(xb, W1_ref[0].astype(jnp.bfloat16),
                    preferred_element_type=jnp.float32) + b1_ref[0], 0.0)
        y = jnp.dot(h.astype(jnp.bfloat16), W2_ref[0].astype(jnp.bfloat16),
                    preferred_element_type=jnp.float32)

        @pl.when(j == 0)
        def _():
            out_ref[...] = y + b2_ref[0]

        @pl.when(j != 0)
        def _():
            out_ref[...] += y


def _grouped_mlp(xs, W1, b1, W2, b2, block_expert, block_valid, block_x):
    grid_spec = pltpu.PrefetchScalarGridSpec(
        num_scalar_prefetch=3,
        grid=(NB, NJ),
        in_specs=[
            pl.BlockSpec((BT, D), lambda b, j, be, v, bx: (bx[b], 0)),
            pl.BlockSpec((1, D, H2), lambda b, j, be, v, bx: (be[b], 0, j)),
            pl.BlockSpec((1, 1, H2), lambda b, j, be, v, bx: (be[b], 0, j)),
            pl.BlockSpec((1, H2, O), lambda b, j, be, v, bx: (be[b], j, 0)),
            pl.BlockSpec((1, 1, O), lambda b, j, be, v, bx: (be[b], 0, 0)),
        ],
        out_specs=pl.BlockSpec((BT, O), lambda b, j, be, v, bx: (b, 0)),
    )
    return pl.pallas_call(
        _grouped_body,
        grid_spec=grid_spec,
        out_shape=jax.ShapeDtypeStruct((P, O), jnp.float32),
    )(block_expert, block_valid, block_x, xs, W1, b1.reshape(E, 1, H), W2,
      b2.reshape(E, 1, O))


def _combine_body(g0_ref, g1_ref, tw_ref, out_ref):
    w0 = tw_ref[:, 0:1]
    w1 = tw_ref[:, 1:2]
    out_ref[...] = w0 * g0_ref[...] + w1 * g1_ref[...]


def _combine(g, top_w):
    btc = 512
    nblk = T // btc
    return pl.pallas_call(
        _combine_body,
        grid=(nblk,),
        in_specs=[pl.BlockSpec((btc, O), lambda t: (t, 0)),
                  pl.BlockSpec((btc, O), lambda t, n=nblk: (t + n, 0)),
                  pl.BlockSpec((btc, K), lambda t: (t, 0))],
        out_specs=pl.BlockSpec((btc, O), lambda t: (t, 0)),
        out_shape=jax.ShapeDtypeStruct((T, O), jnp.float32),
    )(g, g, top_w)


@jax.jit
def kernel(x, Wg, bg, W1, b1, W2, b2):
    top_i, top_w = _gating(x, Wg, bg)
    dest, block_expert, block_valid, block_x = _route_meta(top_i)
    xs = _sc_dispatch_x(x, dest)
    ys = _grouped_mlp(xs, W1, b1, W2, b2, block_expert, block_valid, block_x)
    g = _sc_gather_y(ys, dest)
    return _combine(g, top_w)


# speedup vs baseline: 1.2056x; 1.0090x over previous
import functools

import jax
import jax.numpy as jnp
from jax import lax
from jax.experimental import pallas as pl
from jax.experimental.pallas import tpu as pltpu
from jax.experimental.pallas import tpu_sc as plsc

T = 2048
D = 1024
H = 2048
O = 1024
E = 8
K = 2

BT = 512
NE = T * K
NB = (NE + E * (BT - 1) + BT - 1) // BT
P = NB * BT

NWORKERS = 32


def _gating_body(x_ref, Wg_ref, bg_ref, ti_ref, tw_ref):
    logits = jnp.dot(x_ref[...], Wg_ref[...],
                     preferred_element_type=jnp.float32) + bg_ref[...]
    idx = lax.broadcasted_iota(jnp.int32, (T, E), 1)
    m1 = jnp.max(logits, axis=1, keepdims=True)
    i1 = jnp.min(jnp.where(logits >= m1, idx, E), axis=1, keepdims=True)
    rest = jnp.where(idx == i1, -jnp.inf, logits)
    m2 = jnp.max(rest, axis=1, keepdims=True)
    i2 = jnp.min(jnp.where(rest >= m2, idx, E), axis=1, keepdims=True)
    r = jnp.exp(m2 - m1)
    w1 = 1.0 / (1.0 + r)
    ti_ref[...] = jnp.concatenate([i1, i2], axis=1)
    tw_ref[...] = jnp.concatenate([w1, 1.0 - w1], axis=1)


def _gating(x, Wg, bg):
    return pl.pallas_call(
        _gating_body,
        out_shape=(jax.ShapeDtypeStruct((T, K), jnp.int32),
                   jax.ShapeDtypeStruct((T, K), jnp.float32)),
    )(x, Wg, bg.reshape(1, E))


def _route_meta(top_i):
    f = top_i.T.reshape(-1)
    oh = (f[:, None] == jnp.arange(E, dtype=jnp.int32)[None, :])
    ohi = oh.astype(jnp.int32)
    ranks = jnp.cumsum(ohi, axis=0) - ohi
    rank = jnp.sum(ranks * ohi, axis=1)
    counts = jnp.sum(ohi, axis=0)
    nblk = (counts + BT - 1) // BT
    blk_end = jnp.cumsum(nblk)
    blk_start = jnp.concatenate([jnp.zeros((1,), jnp.int32), blk_end[:-1]])
    dest = (blk_start[f] * BT + rank).astype(jnp.int32)

    nb_used = blk_end[E - 1]
    bidx = jnp.arange(NB, dtype=jnp.int32)
    be = jnp.searchsorted(blk_end, bidx, side="right").astype(jnp.int32)
    be_last = be[nb_used - 1]
    block_expert = jnp.where(bidx < nb_used, be, be_last)
    block_valid = (bidx < nb_used).astype(jnp.int32)
    block_x = jnp.where(bidx < nb_used, bidx, nb_used - 1)
    return dest, block_expert, block_valid, block_x


def _make_sc_gather(n_rows, d, chunk):
    per_w = n_rows // NWORKERS
    n_chunks = per_w // chunk
    mesh = plsc.VectorSubcoreMesh(core_axis_name="c", subcore_axis_name="s")

    def body(src_hbm, idx_hbm, out_hbm, idx_v, rows_v, sem):
        wid = lax.axis_index("s") * 2 + lax.axis_index("c")
        base = wid * per_w

        def step(i, carry):
            off = base + i * chunk
            pltpu.sync_copy(idx_hbm.at[pl.ds(off, chunk)], idx_v)
            pltpu.async_copy(src_hbm.at[idx_v], rows_v, sem).wait()
            pltpu.sync_copy(rows_v, out_hbm.at[pl.ds(off, chunk)])
            return carry

        lax.fori_loop(0, n_chunks, step, 0)

    return pl.kernel(
        body,
        out_type=jax.ShapeDtypeStruct((n_rows, d), jnp.float32),
        mesh=mesh,
        scratch_types=[
            pltpu.VMEM((chunk,), jnp.int32),
            pltpu.VMEM((chunk, d), jnp.float32),
            pltpu.SemaphoreType.DMA,
        ],
    )


@functools.cache
def _sc_gather(n_rows, d, chunk):
    return _make_sc_gather(n_rows, d, chunk)


def _sc_gather_y(src, idx):
    return _sc_gather(NE, O, 64)(src, idx)


def _make_sc_dispatch(chunk):
    per_w = NE // NWORKERS
    n_chunks = per_w // chunk
    mesh = plsc.VectorSubcoreMesh(core_axis_name="c", subcore_axis_name="s")

    def body(x_hbm, dest_hbm, xs_hbm, dest_v, rows_v, sem):
        wid = lax.axis_index("s") * 2 + lax.axis_index("c")
        base = wid * per_w

        def step(i, carry):
            off = base + i * chunk
            tok0 = off % T
            pltpu.sync_copy(dest_hbm.at[pl.ds(off, chunk)], dest_v)
            pltpu.sync_copy(x_hbm.at[pl.ds(tok0, chunk)], rows_v)
            pltpu.async_copy(rows_v, xs_hbm.at[dest_v], sem).wait()
            return carry

        lax.fori_loop(0, n_chunks, step, 0)

    return pl.kernel(
        body,
        out_type=jax.ShapeDtypeStruct((P, D), jnp.float32),
        mesh=mesh,
        scratch_types=[
            pltpu.VMEM((chunk,), jnp.int32),
            pltpu.VMEM((chunk, D), jnp.float32),
            pltpu.SemaphoreType.DMA,
        ],
    )


@functools.cache
def _sc_dispatch(chunk):
    return _make_sc_dispatch(chunk)


def _sc_dispatch_x(x, dest):
    return _sc_dispatch(64)(x, dest)


NJ = 1
H2 = H // NJ


def _grouped_body(be_ref, valid_ref, bx_ref, xs_ref, W1_ref, b1_ref, W2_ref,
                  b2_ref, out_ref):
    b = pl.program_id(0)
    j = pl.program_id(1)

    @pl.when(valid_ref[b] == 1)
    def _():
        h = jnp.maximum(
            jnp.dot(xs_ref[...], W1_ref[0],
                    preferred_element_type=jnp.float32) + b1_ref[0], 0.0)
        y = jnp.dot(h, W2_ref[0], preferred_element_type=jnp.float32)

        @pl.when(j == 0)
        def _():
            out_ref[...] = y + b2_ref[0]

        @pl.when(j != 0)
        def _():
            out_ref[...] += y


def _grouped_mlp(xs, W1, b1, W2, b2, block_expert, block_valid, block_x):
    grid_spec = pltpu.PrefetchScalarGridSpec(
        num_scalar_prefetch=3,
        grid=(NB, NJ),
        in_specs=[
            pl.BlockSpec((BT, D), lambda b, j, be, v, bx: (bx[b], 0)),
            pl.BlockSpec((1, D, H2), lambda b, j, be, v, bx: (be[b], 0, j)),
            pl.BlockSpec((1, 1, H2), lambda b, j, be, v, bx: (be[b], 0, j)),
            pl.BlockSpec((1, H2, O), lambda b, j, be, v, bx: (be[b], j, 0)),
            pl.BlockSpec((1, 1, O), lambda b, j, be, v, bx: (be[b], 0, 0)),
        ],
        out_specs=pl.BlockSpec((BT, O), lambda b, j, be, v, bx: (b, 0)),
    )
    return pl.pallas_call(
        _grouped_body,
        grid_spec=grid_spec,
        out_shape=jax.ShapeDtypeStruct((P, O), jnp.float32),
    )(block_expert, block_valid, block_x, xs, W1, b1.reshape(E, 1, H), W2,
      b2.reshape(E, 1, O))


def _combine_body(g0_ref, g1_ref, tw_ref, out_ref):
    w0 = tw_ref[:, 0:1]
    w1 = tw_ref[:, 1:2]
    out_ref[...] = w0 * g0_ref[...] + w1 * g1_ref[...]


def _combine(g, top_w):
    btc = 512
    nblk = T // btc
    return pl.pallas_call(
        _combine_body,
        grid=(nblk,),
        in_specs=[pl.BlockSpec((btc, O), lambda t: (t, 0)),
                  pl.BlockSpec((btc, O), lambda t, n=nblk: (t + n, 0)),
                  pl.BlockSpec((btc, K), lambda t: (t, 0))],
        out_specs=pl.BlockSpec((btc, O), lambda t: (t, 0)),
        out_shape=jax.ShapeDtypeStruct((T, O), jnp.float32),
    )(g, g, top_w)


@jax.jit
def kernel(x, Wg, bg, W1, b1, W2, b2):
    top_i, top_w = _gating(x, Wg, bg)
    dest, block_expert, block_valid, block_x = _route_meta(top_i)
    xs = _sc_dispatch_x(x, dest)
    ys = _grouped_mlp(xs, W1, b1, W2, b2, block_expert, block_valid, block_x)
    g = _sc_gather_y(ys, dest)
    return _combine(g, top_w)


# speedup vs baseline: 1.2367x; 1.0258x over previous
import functools

import jax
import jax.numpy as jnp
from jax import lax
from jax.experimental import pallas as pl
from jax.experimental.pallas import tpu as pltpu
from jax.experimental.pallas import tpu_sc as plsc

T = 2048
D = 1024
H = 2048
O = 1024
E = 8
K = 2

BT = 512
NE = T * K
NB = (NE + E * (BT - 1) + BT - 1) // BT
P = NB * BT

NWORKERS = 32


def _colcumsum_excl(a):
    c = a
    d = 1
    while d < T:
        c = c + jnp.concatenate(
            [jnp.zeros((d, E), jnp.float32), c[:T - d]], axis=0)
        d *= 2
    return c - a


def _gate_route_body(x_ref, Wg_ref, bg_ref, tw_ref, d0_ref, d1_ref,
                     bee_ref, val_ref, bx_ref):
    logits = jnp.dot(x_ref[...], Wg_ref[...],
                     preferred_element_type=jnp.float32) + bg_ref[...]
    idx = lax.broadcasted_iota(jnp.int32, (T, E), 1)
    m1 = jnp.max(logits, axis=1, keepdims=True)
    i1 = jnp.min(jnp.where(logits >= m1, idx, E), axis=1, keepdims=True)
    oh1 = idx == i1
    rest = jnp.where(oh1, -jnp.inf, logits)
    m2 = jnp.max(rest, axis=1, keepdims=True)
    i2 = jnp.min(jnp.where(rest >= m2, idx, E), axis=1, keepdims=True)
    oh2 = idx == i2
    r = jnp.exp(m2 - m1)
    w1 = 1.0 / (1.0 + r)
    tw_ref[...] = jnp.concatenate([w1, 1.0 - w1], axis=1)

    o1 = oh1.astype(jnp.float32)
    o2 = oh2.astype(jnp.float32)
    c1 = _colcumsum_excl(o1)
    tot1 = jnp.sum(o1, axis=0, keepdims=True)
    c2 = _colcumsum_excl(o2) + tot1
    tot = tot1 + jnp.sum(o2, axis=0, keepdims=True)
    rank1 = jnp.sum(c1 * o1, axis=1, keepdims=True)
    rank2 = jnp.sum(c2 * o2, axis=1, keepdims=True)
    nblk = jnp.floor((tot + (BT - 1)) / BT)
    nblk_b = jnp.broadcast_to(nblk, (T, E))
    bs1 = jnp.sum(jnp.where(idx < i1, nblk_b, 0.0), axis=1, keepdims=True)
    bs2 = jnp.sum(jnp.where(idx < i2, nblk_b, 0.0), axis=1, keepdims=True)
    d0_ref[...] = (bs1 * BT + rank1).astype(jnp.int32)
    d1_ref[...] = (bs2 * BT + rank2).astype(jnp.int32)

    blk_end = nblk
    d = 1
    while d < E:
        blk_end = blk_end + jnp.concatenate(
            [jnp.zeros((1, d), jnp.float32), blk_end[:, :E - d]], axis=1)
        d *= 2
    bidx = lax.broadcasted_iota(jnp.int32, (NB, 1), 0).astype(jnp.float32)
    be = jnp.sum((bidx >= blk_end).astype(jnp.float32),
                 axis=1, keepdims=True)
    nb_used = jnp.sum(nblk, axis=1, keepdims=True)
    valid = bidx < nb_used
    be_last = jnp.sum(jnp.where(bidx == nb_used - 1.0, be, 0.0), axis=0,
                      keepdims=True)
    bee_ref[...] = jnp.where(valid, be, be_last).astype(jnp.int32)
    val_ref[...] = valid.astype(jnp.int32)
    bx_ref[...] = jnp.minimum(bidx, nb_used - 1.0).astype(jnp.int32)


def _gate_route(x, Wg, bg):
    tw, d0, d1, bee, val, bx = pl.pallas_call(
        _gate_route_body,
        out_shape=(jax.ShapeDtypeStruct((T, K), jnp.float32),
                   jax.ShapeDtypeStruct((T, 1), jnp.int32),
                   jax.ShapeDtypeStruct((T, 1), jnp.int32),
                   jax.ShapeDtypeStruct((NB, 1), jnp.int32),
                   jax.ShapeDtypeStruct((NB, 1), jnp.int32),
                   jax.ShapeDtypeStruct((NB, 1), jnp.int32)),
    )(x, Wg, bg.reshape(1, E))
    dest = jnp.concatenate([d0.reshape(-1), d1.reshape(-1)])
    return tw, dest, bee.reshape(-1), val.reshape(-1), bx.reshape(-1)


def _make_sc_gather(n_rows, d, chunk):
    per_w = n_rows // NWORKERS
    n_chunks = per_w // chunk
    mesh = plsc.VectorSubcoreMesh(core_axis_name="c", subcore_axis_name="s")

    def body(src_hbm, idx_hbm, out_hbm, idx_v, rows_v, sem):
        wid = lax.axis_index("s") * 2 + lax.axis_index("c")
        base = wid * per_w

        def step(i, carry):
            off = base + i * chunk
            pltpu.sync_copy(idx_hbm.at[pl.ds(off, chunk)], idx_v)
            pltpu.async_copy(src_hbm.at[idx_v], rows_v, sem).wait()
            pltpu.sync_copy(rows_v, out_hbm.at[pl.ds(off, chunk)])
            return carry

        lax.fori_loop(0, n_chunks, step, 0)

    return pl.kernel(
        body,
        out_type=jax.ShapeDtypeStruct((n_rows, d), jnp.float32),
        mesh=mesh,
        scratch_types=[
            pltpu.VMEM((chunk,), jnp.int32),
            pltpu.VMEM((chunk, d), jnp.float32),
            pltpu.SemaphoreType.DMA,
        ],
    )


@functools.cache
def _sc_gather(n_rows, d, chunk):
    return _make_sc_gather(n_rows, d, chunk)


def _sc_gather_y(src, idx):
    return _sc_gather(NE, O, 64)(src, idx)


def _make_sc_dispatch(chunk):
    per_w = NE // NWORKERS
    n_chunks = per_w // chunk
    mesh = plsc.VectorSubcoreMesh(core_axis_name="c", subcore_axis_name="s")

    def body(x_hbm, dest_hbm, xs_hbm, dest_v, rows_v, sem):
        wid = lax.axis_index("s") * 2 + lax.axis_index("c")
        base = wid * per_w

        def step(i, carry):
            off = base + i * chunk
            tok0 = off % T
            pltpu.sync_copy(dest_hbm.at[pl.ds(off, chunk)], dest_v)
            pltpu.sync_copy(x_hbm.at[pl.ds(tok0, chunk)], rows_v)
            pltpu.async_copy(rows_v, xs_hbm.at[dest_v], sem).wait()
            return carry

        lax.fori_loop(0, n_chunks, step, 0)

    return pl.kernel(
        body,
        out_type=jax.ShapeDtypeStruct((P, D), jnp.float32),
        mesh=mesh,
        scratch_types=[
            pltpu.VMEM((chunk,), jnp.int32),
            pltpu.VMEM((chunk, D), jnp.float32),
            pltpu.SemaphoreType.DMA,
        ],
    )


@functools.cache
def _sc_dispatch(chunk):
    return _make_sc_dispatch(chunk)


def _sc_dispatch_x(x, dest):
    return _sc_dispatch(64)(x, dest)


NJ = 1
H2 = H // NJ


def _grouped_body(be_ref, valid_ref, bx_ref, xs_ref, W1_ref, b1_ref, W2_ref,
                  b2_ref, out_ref):
    b = pl.program_id(0)
    j = pl.program_id(1)

    @pl.when(valid_ref[b] == 1)
    def _():
        h = jnp.maximum(
            jnp.dot(xs_ref[...], W1_ref[0],
                    preferred_element_type=jnp.float32) + b1_ref[0], 0.0)
        y = jnp.dot(h, W2_ref[0], preferred_element_type=jnp.float32)

        @pl.when(j == 0)
        def _():
            out_ref[...] = y + b2_ref[0]

        @pl.when(j != 0)
        def _():
            out_ref[...] += y


def _grouped_mlp(xs, W1, b1, W2, b2, block_expert, block_valid, block_x):
    grid_spec = pltpu.PrefetchScalarGridSpec(
        num_scalar_prefetch=3,
        grid=(NB, NJ),
        in_specs=[
            pl.BlockSpec((BT, D), lambda b, j, be, v, bx: (bx[b], 0)),
            pl.BlockSpec((1, D, H2), lambda b, j, be, v, bx: (be[b], 0, j)),
            pl.BlockSpec((1, 1, H2), lambda b, j, be, v, bx: (be[b], 0, j)),
            pl.BlockSpec((1, H2, O), lambda b, j, be, v, bx: (be[b], j, 0)),
            pl.BlockSpec((1, 1, O), lambda b, j, be, v, bx: (be[b], 0, 0)),
        ],
        out_specs=pl.BlockSpec((BT, O), lambda b, j, be, v, bx: (b, 0)),
    )
    return pl.pallas_call(
        _grouped_body,
        grid_spec=grid_spec,
        out_shape=jax.ShapeDtypeStruct((P, O), jnp.float32),
    )(block_expert, block_valid, block_x, xs, W1, b1.reshape(E, 1, H), W2,
      b2.reshape(E, 1, O))


def _combine_body(g0_ref, g1_ref, tw_ref, out_ref):
    w0 = tw_ref[:, 0:1]
    w1 = tw_ref[:, 1:2]
    out_ref[...] = w0 * g0_ref[...] + w1 * g1_ref[...]


def _combine(g, top_w):
    btc = 512
    nblk = T // btc
    return pl.pallas_call(
        _combine_body,
        grid=(nblk,),
        in_specs=[pl.BlockSpec((btc, O), lambda t: (t, 0)),
                  pl.BlockSpec((btc, O), lambda t, n=nblk: (t + n, 0)),
                  pl.BlockSpec((btc, K), lambda t: (t, 0))],
        out_specs=pl.BlockSpec((btc, O), lambda t: (t, 0)),
        out_shape=jax.ShapeDtypeStruct((T, O), jnp.float32),
    )(g, g, top_w)


@jax.jit
def kernel(x, Wg, bg, W1, b1, W2, b2):
    top_w, dest, block_expert, block_valid, block_x = _gate_route(x, Wg, bg)
    xs = _sc_dispatch_x(x, dest)
    ys = _grouped_mlp(xs, W1, b1, W2, b2, block_expert, block_valid, block_x)
    g = _sc_gather_y(ys, dest)
    return _combine(g, top_w)
